# baseline (speedup 1.0000x reference)
import jax
import jax.numpy as jnp
from jax import lax
from jax.experimental import pallas as pl
from jax.experimental.pallas import tpu as pltpu

_TILE = 4096
_CHUNK = 64


def _round_up(x: int, m: int) -> int:
    return (x + m - 1) // m * m


def _gmf_gather_kernel(ids_ref, u_tbl_ref, v_tbl_ref, out_ref):
    def chunk_body(c, carry):
        base = c * _CHUNK
        u_rows = []
        v_rows = []
        for j in range(_CHUNK):
            w = ids_ref[0, 0, base + j]
            u_rows.append(u_tbl_ref[w & 4095, 0])
            v_rows.append(v_tbl_ref[w >> 12, 0])
        for j in range(_CHUNK):
            out_ref[pl.ds(base + j, 1), 0] = (u_rows[j] * v_rows[j])[None]
        return carry

    lax.fori_loop(0, _TILE // _CHUNK, chunk_body, 0)


@jax.jit
def kernel(u_idx, v_idx, u_table, v_table):
    batch = int(u_idx.shape[0])
    nu, emb = u_table.shape
    ni, emb_v = v_table.shape
    assert emb == emb_v, "embedding dims must match"
    out_dtype = jnp.result_type(u_table.dtype, v_table.dtype)

    u_idx = jnp.clip(u_idx.astype(jnp.int32), 0, nu - 1)
    v_idx = jnp.clip(v_idx.astype(jnp.int32), 0, ni - 1)
    packed = u_idx | (v_idx << 12)

    batch_pad = _round_up(batch, _TILE)
    if batch_pad != batch:
        packed = jnp.pad(packed, (0, batch_pad - batch))
    n_tiles = batch_pad // _TILE

    ids = packed.reshape(n_tiles, 1, _TILE)
    u_t3 = u_table.reshape(nu, 1, emb)
    v_t3 = v_table.reshape(ni, 1, emb)

    out = pl.pallas_call(
        _gmf_gather_kernel,
        out_shape=jax.ShapeDtypeStruct((batch_pad, 1, emb), out_dtype),
        grid=(n_tiles,),
        in_specs=[
            pl.BlockSpec((1, 1, _TILE), lambda i: (i, 0, 0),
                         memory_space=pltpu.SMEM),
            pl.BlockSpec((nu, 1, emb), lambda i: (0, 0, 0)),
            pl.BlockSpec((ni, 1, emb), lambda i: (0, 0, 0)),
        ],
        out_specs=pl.BlockSpec((_TILE, 1, emb), lambda i: (i, 0, 0)),
        compiler_params=pltpu.CompilerParams(
            dimension_semantics=("parallel",),
            vmem_limit_bytes=56 * 1024 * 1024,
        ),
    )(ids, u_t3, v_t3)

    return out.reshape(batch_pad, emb)[:batch]
